# Initial kernel scaffold; baseline (speedup 1.0000x reference)
import jax
import jax.numpy as jnp
from jax.experimental import pallas as pl


def kernel(token_ids, base_table, special_table):
    raise NotImplementedError("write your pallas kernel here")


# speedup vs baseline: 3.6773x; 3.6773x over previous
import functools

import jax
import jax.numpy as jnp
from jax import lax
from jax.experimental import pallas as pl
from jax.experimental.pallas import tpu as pltpu
from jax.experimental.pallas import tpu_sc as plsc

BASE_VOCAB = 100000
N_SPECIAL = 16
HIDDEN = 64
COMBINED = BASE_VOCAB + N_SPECIAL

N_TOKENS = 16384 * 50

NW = 32
GATHER = 128
GPC = 4
CHUNK = GATHER * GPC
PER_W = N_TOKENS // NW
N_CHUNK = PER_W // CHUNK
IDX_ROWS_PER_W = PER_W // GATHER


def _concat_body(base_ref, special_ref, out_ref, sem0, sem1):
    c0 = pltpu.make_async_copy(base_ref, out_ref.at[pl.ds(0, BASE_VOCAB)], sem0)
    c1 = pltpu.make_async_copy(
        special_ref, out_ref.at[pl.ds(BASE_VOCAB, N_SPECIAL)], sem1)
    c0.start()
    c1.start()
    c0.wait()
    c1.wait()


def _build_combined(base, special):
    return pl.pallas_call(
        _concat_body,
        out_shape=jax.ShapeDtypeStruct((COMBINED, HIDDEN), jnp.float32),
        in_specs=[
            pl.BlockSpec(memory_space=pltpu.HBM),
            pl.BlockSpec(memory_space=pltpu.HBM),
        ],
        out_specs=pl.BlockSpec(memory_space=pltpu.HBM),
        scratch_shapes=[pltpu.SemaphoreType.DMA, pltpu.SemaphoreType.DMA],
    )(base, special)


_mesh = plsc.VectorSubcoreMesh(core_axis_name="c", subcore_axis_name="s")


@functools.partial(
    pl.kernel,
    out_type=jax.ShapeDtypeStruct((N_TOKENS, HIDDEN), jnp.float32),
    mesh=_mesh,
    compiler_params=pltpu.CompilerParams(use_tc_tiling_on_sc=False),
    scratch_types=[
        pltpu.VMEM((2, GPC, GATHER), jnp.int32),
        pltpu.VMEM((2, CHUNK, HIDDEN), jnp.float32),
        pltpu.SemaphoreType.DMA,
        pltpu.SemaphoreType.DMA,
        pltpu.SemaphoreType.DMA,
        pltpu.SemaphoreType.DMA,
    ],
)
def _gather_kernel(table_hbm, idx_hbm, out_hbm, idx_v, rows_v,
                   sg0, sg1, so0, so1):
    wid = lax.axis_index("s") * 2 + lax.axis_index("c")
    idx_base = wid * IDX_ROWS_PER_W
    out_base = wid * PER_W
    sgs = (sg0, sg1)
    sos = (so0, so1)

    def do_chunk(g, b, first):
        @pl.when(jnp.logical_not(first))
        def _():
            pltpu.make_async_copy(
                rows_v.at[b], out_hbm.at[pl.ds(out_base, CHUNK)], sos[b]
            ).wait()

        pltpu.sync_copy(idx_hbm.at[pl.ds(idx_base + g * GPC, GPC)],
                        idx_v.at[b])
        copies = []
        for j in range(GPC):
            copies.append(pltpu.async_copy(
                table_hbm.at[idx_v.at[b, j]],
                rows_v.at[b, pl.ds(j * GATHER, GATHER)],
                sgs[b]))
        for cp in copies:
            cp.wait()
        pltpu.async_copy(rows_v.at[b],
                         out_hbm.at[pl.ds(out_base + g * CHUNK, CHUNK)],
                         sos[b])

    def body(i, carry):
        do_chunk(2 * i, 0, i == 0)
        do_chunk(2 * i + 1, 1, i == 0)
        return carry

    lax.fori_loop(0, N_CHUNK // 2, body, 0)

    for b in range(2):
        pltpu.make_async_copy(
            rows_v.at[b], out_hbm.at[pl.ds(out_base, CHUNK)], sos[b]
        ).wait()


def kernel(token_ids, base_table, special_table):
    combined = _build_combined(base_table, special_table)
    idx = token_ids.reshape(N_TOKENS // GATHER, GATHER).astype(jnp.int32)
    out = _gather_kernel(combined, idx)
    return out.reshape(16384, 50, HIDDEN)


# speedup vs baseline: 11.2777x; 3.0668x over previous
import functools

import jax
import jax.numpy as jnp
from jax import lax
from jax.experimental import pallas as pl
from jax.experimental.pallas import tpu as pltpu
from jax.experimental.pallas import tpu_sc as plsc

BASE_VOCAB = 100000
N_SPECIAL = 16
HIDDEN = 64

N_TOKENS = 16384 * 50

NW = 32
GATHER = 128
GPC = 4
CHUNK = GATHER * GPC
PER_W = N_TOKENS // NW
N_CHUNK = PER_W // CHUNK
N_GROUP = CHUNK // 16

_mesh = plsc.VectorSubcoreMesh(core_axis_name="c", subcore_axis_name="s")


@functools.partial(
    pl.kernel,
    out_type=jax.ShapeDtypeStruct((N_TOKENS, HIDDEN), jnp.float32),
    mesh=_mesh,
    compiler_params=pltpu.CompilerParams(use_tc_tiling_on_sc=False,
                                         needs_layout_passes=False),
    scratch_types=[
        pltpu.VMEM((N_SPECIAL, HIDDEN), jnp.float32),
        pltpu.VMEM((2, CHUNK), jnp.int32),
        pltpu.VMEM((2, CHUNK), jnp.int32),
        pltpu.VMEM((2, CHUNK, HIDDEN), jnp.float32),
        pltpu.SemaphoreType.DMA,
        pltpu.SemaphoreType.DMA,
        pltpu.SemaphoreType.DMA,
        pltpu.SemaphoreType.DMA,
    ],
)
def _lookup_kernel(base_hbm, special_hbm, idx_hbm, out_hbm,
                   spec_v, idx_v, gidx_v, rows_v, sg0, sg1, so0, so1):
    wid = lax.axis_index("s") * 2 + lax.axis_index("c")
    tok_base = wid * PER_W
    sgs = (sg0, sg1)
    sos = (so0, so1)

    pltpu.sync_copy(special_hbm, spec_v)
    lanes = lax.iota(jnp.int32, 16)

    def do_chunk(g, b, first):
        @pl.when(jnp.logical_not(first))
        def _():
            pltpu.make_async_copy(
                rows_v.at[b], out_hbm.at[pl.ds(tok_base, CHUNK)], sos[b]
            ).wait()

        start = tok_base + g * CHUNK
        pltpu.sync_copy(idx_hbm.at[pl.ds(start, CHUNK)], idx_v.at[b])

        acc = jnp.zeros((16,), jnp.int32)
        for k in range(N_GROUP):
            v = idx_v.at[b][pl.ds(k * 16, 16)]
            m = v >= BASE_VOCAB
            gidx_v.at[b][pl.ds(k * 16, 16)] = jnp.where(m, 0, v)
            acc = acc + m.astype(jnp.int32)
        has_special = plsc.cumsum(acc)[15] > 0

        copies = []
        for j in range(GPC):
            copies.append(pltpu.async_copy(
                base_hbm.at[gidx_v.at[b, pl.ds(j * GATHER, GATHER)]],
                rows_v.at[b, pl.ds(j * GATHER, GATHER)],
                sgs[b]))
        for cp in copies:
            cp.wait()

        rows_chunk = rows_v.at[b]

        @pl.when(has_special)
        def _():
            def fix_group(k, carry):
                v = idx_v.at[b][pl.ds(k * 16, 16)]
                m = v >= BASE_VOCAB
                anyg = plsc.cumsum(m.astype(jnp.int32))[15] > 0

                @pl.when(anyg)
                def _():
                    sid = jnp.where(m, v - BASE_VOCAB, 0)
                    rowi = k * 16 + lanes

                    def fix_col(col, carry2):
                        colv = jnp.full((16,), col, jnp.int32)
                        vals = plsc.load_gather(spec_v, [sid, colv])
                        plsc.store_scatter(rows_chunk, [rowi, colv], vals,
                                           mask=m)
                        return carry2

                    lax.fori_loop(0, HIDDEN, fix_col, 0)

                return carry

            lax.fori_loop(0, N_GROUP, fix_group, 0)

        pltpu.async_copy(rows_v.at[b],
                         out_hbm.at[pl.ds(start, CHUNK)],
                         sos[b])

    def body(i, carry):
        do_chunk(2 * i, 0, i == 0)
        do_chunk(2 * i + 1, 1, i == 0)
        return carry

    lax.fori_loop(0, N_CHUNK // 2, body, 0)

    for b in range(2):
        pltpu.make_async_copy(
            rows_v.at[b], out_hbm.at[pl.ds(tok_base, CHUNK)], sos[b]
        ).wait()


def kernel(token_ids, base_table, special_table):
    idx = token_ids.reshape(N_TOKENS).astype(jnp.int32)
    out = _lookup_kernel(base_table, special_table, idx)
    return out.reshape(16384, 50, HIDDEN)
